# baseline (speedup 1.0000x reference)
import functools

import jax
import jax.numpy as jnp
from jax import lax
from jax.experimental import pallas as pl
from jax.experimental.pallas import tpu as pltpu
from jax.experimental.pallas import tpu_sc as plsc

N = 10000
NPAD = 10240
E = 160000
NC = 2
NS = 16
B = 128
NBT = 80
EPAD = NS * NBT * B
HALF = NPAD // NC
ACC_R = HALF + 8
RPT = HALF // NS
NCHUNK = 4
RB = 400
GRID_R = N // RB

_mesh = plsc.VectorSubcoreMesh(core_axis_name="c", subcore_axis_name="s")


def _redirect_cols(cols_v, sc):
    base = sc * HALF

    def body(i, carry):
        for k in range(B // 16):
            c = cols_v[i, pl.ds(k * 16, 16)] - base
            ok = (c >= 0) & (c < HALF)
            cols_v[i, pl.ds(k * 16, 16)] = jnp.where(ok, c, HALF)
        return carry

    lax.fori_loop(0, NBT, body, 0)


def _deg_body(cols_hbm, ones_hbm, zeros_hbm, deg_hbm, cols_v, ones_v, dacc):
    sc = lax.axis_index("c")
    tid = lax.axis_index("s")
    pltpu.sync_copy(cols_hbm.at[tid], cols_v)
    pltpu.sync_copy(ones_hbm, ones_v)
    pltpu.sync_copy(zeros_hbm, dacc.at[pl.ds(tid * RPT, RPT)])
    _redirect_cols(cols_v, sc)
    plsc.subcore_barrier()

    def body(i, carry):
        pltpu.sync_copy(ones_v, dacc.at[cols_v.at[i]], add=True)
        return carry

    lax.fori_loop(0, NBT, body, 0)
    plsc.subcore_barrier()
    pltpu.sync_copy(dacc.at[pl.ds(tid * RPT, RPT)],
                    deg_hbm.at[pl.ds(sc * HALF + tid * RPT, RPT)])
    return None


_sc_deg = functools.partial(
    pl.kernel,
    out_type=jax.ShapeDtypeStruct((NPAD, 128), jnp.float32),
    mesh=_mesh,
    scratch_types=[
        pltpu.VMEM((NBT, B), jnp.int32),
        pltpu.VMEM((B, 128), jnp.float32),
        pltpu.VMEM_SHARED((ACC_R, 128), jnp.float32),
    ],
)(_deg_body)


def _scatter_body(rows_hbm, cols_hbm, y_hbm, out_hbm,
                  rows_v, cols_v, g0, g1, acc, s0, s1):
    sc = lax.axis_index("c")
    tid = lax.axis_index("s")
    pltpu.sync_copy(rows_hbm.at[tid], rows_v)
    pltpu.sync_copy(cols_hbm.at[tid], cols_v)
    _redirect_cols(cols_v, sc)
    for j in range(NCHUNK):
        ytab = y_hbm.at[j]
        pltpu.sync_copy(ytab.at[pl.ds(sc * HALF + tid * RPT, RPT)],
                        acc.at[pl.ds(tid * RPT, RPT)])
        plsc.subcore_barrier()

        pltpu.async_copy(ytab.at[rows_v.at[0]], g0, s0)

        def body(i, carry):
            c1 = pltpu.async_copy(ytab.at[rows_v.at[2 * i + 1]], g1, s1)
            pltpu.make_async_copy(ytab.at[rows_v.at[2 * i]], g0, s0).wait()
            pltpu.sync_copy(g0, acc.at[cols_v.at[2 * i]], add=True)
            pltpu.async_copy(ytab.at[rows_v.at[2 * i + 2]], g0, s0)
            c1.wait()
            pltpu.sync_copy(g1, acc.at[cols_v.at[2 * i + 1]], add=True)
            return carry

        lax.fori_loop(0, NBT // 2 - 1, body, 0)
        c1 = pltpu.async_copy(ytab.at[rows_v.at[NBT - 1]], g1, s1)
        pltpu.make_async_copy(ytab.at[rows_v.at[NBT - 2]], g0, s0).wait()
        pltpu.sync_copy(g0, acc.at[cols_v.at[NBT - 2]], add=True)
        c1.wait()
        pltpu.sync_copy(g1, acc.at[cols_v.at[NBT - 1]], add=True)

        plsc.subcore_barrier()
        pltpu.sync_copy(acc.at[pl.ds(tid * RPT, RPT)],
                        out_hbm.at[j].at[pl.ds(sc * HALF + tid * RPT, RPT)])
        plsc.subcore_barrier()
    return None


_sc_scatter = functools.partial(
    pl.kernel,
    out_type=jax.ShapeDtypeStruct((NCHUNK, NPAD, 128), jnp.float32),
    mesh=_mesh,
    scratch_types=[
        pltpu.VMEM((NBT, B), jnp.int32),
        pltpu.VMEM((NBT, B), jnp.int32),
        pltpu.VMEM((B, 128), jnp.float32),
        pltpu.VMEM((B, 128), jnp.float32),
        pltpu.VMEM_SHARED((ACC_R, 128), jnp.float32),
        pltpu.SemaphoreType.DMA,
        pltpu.SemaphoreType.DMA,
    ],
)(_scatter_body)


def _dis_block(deg_ref):
    s = jnp.sum(deg_ref[...], axis=1, keepdims=True) * (1.0 / 128.0) + 1.0
    return lax.rsqrt(s)


def _m1_body(x_ref, w_ref, deg_ref, out_ref):
    dis = _dis_block(deg_ref)
    xw = jnp.dot(x_ref[...], w_ref[...], preferred_element_type=jnp.float32)
    y = dis * xw
    for j in range(NCHUNK):
        out_ref[j] = y[:, j * 128:(j + 1) * 128]


def _mmid_body(a_ref, w_ref, b_ref, deg_ref, out_ref):
    dis = _dis_block(deg_ref)
    hcat = jnp.concatenate([a_ref[j] for j in range(NCHUNK)], axis=1)
    h = dis * hcat + b_ref[...]
    xw = jnp.dot(h, w_ref[...], preferred_element_type=jnp.float32)
    y = dis * xw
    for j in range(NCHUNK):
        out_ref[j] = y[:, j * 128:(j + 1) * 128]


def _mfinal_body(a_ref, b_ref, deg_ref, out_ref):
    dis = _dis_block(deg_ref)
    hcat = jnp.concatenate([a_ref[j] for j in range(NCHUNK)], axis=1)
    out_ref[...] = dis * hcat + b_ref[...]


_deg_spec = pl.BlockSpec((RB, 128), lambda r: (r, 0))
_chunk_spec = pl.BlockSpec((NCHUNK, RB, 128), lambda r: (0, r, 0))


def _m1(x, w, deg):
    return pl.pallas_call(
        _m1_body,
        grid=(GRID_R,),
        in_specs=[
            pl.BlockSpec((RB, 256), lambda r: (r, 0)),
            pl.BlockSpec((256, 512), lambda r: (0, 0)),
            _deg_spec,
        ],
        out_specs=_chunk_spec,
        out_shape=jax.ShapeDtypeStruct((NCHUNK, NPAD, 128), jnp.float32),
    )(x, w, deg)


def _mmid(a, w, b, deg):
    return pl.pallas_call(
        _mmid_body,
        grid=(GRID_R,),
        in_specs=[
            _chunk_spec,
            pl.BlockSpec((512, 512), lambda r: (0, 0)),
            pl.BlockSpec((1, 512), lambda r: (0, 0)),
            _deg_spec,
        ],
        out_specs=_chunk_spec,
        out_shape=jax.ShapeDtypeStruct((NCHUNK, NPAD, 128), jnp.float32),
    )(a, w, b, deg)


def _mfinal(a, b, deg):
    return pl.pallas_call(
        _mfinal_body,
        grid=(GRID_R,),
        in_specs=[
            _chunk_spec,
            pl.BlockSpec((1, 512), lambda r: (0, 0)),
            _deg_spec,
        ],
        out_specs=pl.BlockSpec((RB, 512), lambda r: (r, 0)),
        out_shape=jax.ShapeDtypeStruct((N, 512), jnp.float32),
    )(a, b, deg)


def kernel(node_features, edge_index, W1, b1, W2, b2, W3, b3):
    ei = edge_index.astype(jnp.int32)
    pad = EPAD - E
    rows3 = jnp.concatenate(
        [ei[0], jnp.zeros((pad,), jnp.int32)]).reshape(NS, NBT, B)
    cols3 = jnp.concatenate(
        [ei[1], jnp.full((pad,), NPAD, jnp.int32)]).reshape(NS, NBT, B)
    ones = jnp.ones((B, 128), jnp.float32)
    zeros = jnp.zeros((RPT, 128), jnp.float32)

    deg = _sc_deg(cols3, ones, zeros)

    y1 = _m1(node_features, W1, deg)
    a1 = _sc_scatter(rows3, cols3, y1)
    y2 = _mmid(a1, W2, b1.reshape(1, 512), deg)
    a2 = _sc_scatter(rows3, cols3, y2)
    y3 = _mmid(a2, W3, b2.reshape(1, 512), deg)
    a3 = _sc_scatter(rows3, cols3, y3)
    return _mfinal(a3, b3.reshape(1, 512), deg)


# speedup vs baseline: 1.0035x; 1.0035x over previous
import functools

import jax
import jax.numpy as jnp
from jax import lax
from jax.experimental import pallas as pl
from jax.experimental.pallas import tpu as pltpu
from jax.experimental.pallas import tpu_sc as plsc

N = 10000
NPAD = 10240
E = 160000
NC = 2
NS = 16
B = 80
NBT = 128
EPAD = NS * NBT * B
HALF = NPAD // NC
ACC_R = HALF + 8
RPT = HALF // NS
NCHUNK = 4
RB = 400
GRID_R = N // RB

_mesh = plsc.VectorSubcoreMesh(core_axis_name="c", subcore_axis_name="s")


def _redirect_cols(cols_v, sc):
    base = sc * HALF

    def body(i, carry):
        for k in range(B // 16):
            c = cols_v[i, pl.ds(k * 16, 16)] - base
            ok = (c >= 0) & (c < HALF)
            cols_v[i, pl.ds(k * 16, 16)] = jnp.where(ok, c, HALF)
        return carry

    lax.fori_loop(0, NBT, body, 0)


def _deg_body(cols_hbm, ones_hbm, zeros_hbm, deg_hbm, cols_v, ones_v, dacc):
    sc = lax.axis_index("c")
    tid = lax.axis_index("s")
    pltpu.sync_copy(cols_hbm.at[tid], cols_v)
    pltpu.sync_copy(ones_hbm, ones_v)
    pltpu.sync_copy(zeros_hbm, dacc.at[pl.ds(tid * RPT, RPT)])
    _redirect_cols(cols_v, sc)
    plsc.subcore_barrier()

    def body(i, carry):
        pltpu.sync_copy(ones_v, dacc.at[cols_v.at[i]], add=True)
        return carry

    lax.fori_loop(0, NBT, body, 0)
    plsc.subcore_barrier()
    pltpu.sync_copy(dacc.at[pl.ds(tid * RPT, RPT)],
                    deg_hbm.at[pl.ds(sc * HALF + tid * RPT, RPT)])
    return None


_sc_deg = functools.partial(
    pl.kernel,
    out_type=jax.ShapeDtypeStruct((NPAD, 128), jnp.float32),
    mesh=_mesh,
    scratch_types=[
        pltpu.VMEM((NBT, B), jnp.int32),
        pltpu.VMEM((B, 128), jnp.float32),
        pltpu.VMEM_SHARED((ACC_R, 128), jnp.float32),
    ],
)(_deg_body)


def _scatter_body(rows_hbm, cols_hbm, y_hbm, out_hbm,
                  rows_v, cols_v, g0, g1, acc, s0, s1):
    sc = lax.axis_index("c")
    tid = lax.axis_index("s")
    pltpu.sync_copy(rows_hbm.at[tid], rows_v)
    pltpu.sync_copy(cols_hbm.at[tid], cols_v)
    _redirect_cols(cols_v, sc)
    for j in range(NCHUNK):
        ytab = y_hbm.at[j]
        pltpu.sync_copy(ytab.at[pl.ds(sc * HALF + tid * RPT, RPT)],
                        acc.at[pl.ds(tid * RPT, RPT)])
        plsc.subcore_barrier()

        pltpu.async_copy(ytab.at[rows_v.at[0]], g0, s0)

        def body(i, carry):
            c1 = pltpu.async_copy(ytab.at[rows_v.at[2 * i + 1]], g1, s1)
            pltpu.make_async_copy(ytab.at[rows_v.at[2 * i]], g0, s0).wait()
            pltpu.sync_copy(g0, acc.at[cols_v.at[2 * i]], add=True)
            pltpu.async_copy(ytab.at[rows_v.at[2 * i + 2]], g0, s0)
            c1.wait()
            pltpu.sync_copy(g1, acc.at[cols_v.at[2 * i + 1]], add=True)
            return carry

        lax.fori_loop(0, NBT // 2 - 1, body, 0)
        c1 = pltpu.async_copy(ytab.at[rows_v.at[NBT - 1]], g1, s1)
        pltpu.make_async_copy(ytab.at[rows_v.at[NBT - 2]], g0, s0).wait()
        pltpu.sync_copy(g0, acc.at[cols_v.at[NBT - 2]], add=True)
        c1.wait()
        pltpu.sync_copy(g1, acc.at[cols_v.at[NBT - 1]], add=True)

        plsc.subcore_barrier()
        pltpu.sync_copy(acc.at[pl.ds(tid * RPT, RPT)],
                        out_hbm.at[j].at[pl.ds(sc * HALF + tid * RPT, RPT)])
        plsc.subcore_barrier()
    return None


_sc_scatter = functools.partial(
    pl.kernel,
    out_type=jax.ShapeDtypeStruct((NCHUNK, NPAD, 128), jnp.float32),
    mesh=_mesh,
    scratch_types=[
        pltpu.VMEM((NBT, B), jnp.int32),
        pltpu.VMEM((NBT, B), jnp.int32),
        pltpu.VMEM((B, 128), jnp.float32),
        pltpu.VMEM((B, 128), jnp.float32),
        pltpu.VMEM_SHARED((ACC_R, 128), jnp.float32),
        pltpu.SemaphoreType.DMA,
        pltpu.SemaphoreType.DMA,
    ],
)(_scatter_body)


def _dis_block(deg_ref):
    s = jnp.sum(deg_ref[...], axis=1, keepdims=True) * (1.0 / 128.0) + 1.0
    return lax.rsqrt(s)


def _m1_body(x_ref, w_ref, deg_ref, out_ref):
    dis = _dis_block(deg_ref)
    xw = jnp.dot(x_ref[...], w_ref[...], preferred_element_type=jnp.float32)
    y = dis * xw
    for j in range(NCHUNK):
        out_ref[j] = y[:, j * 128:(j + 1) * 128]


def _mmid_body(a_ref, w_ref, b_ref, deg_ref, out_ref):
    dis = _dis_block(deg_ref)
    hcat = jnp.concatenate([a_ref[j] for j in range(NCHUNK)], axis=1)
    h = dis * hcat + b_ref[...]
    xw = jnp.dot(h, w_ref[...], preferred_element_type=jnp.float32)
    y = dis * xw
    for j in range(NCHUNK):
        out_ref[j] = y[:, j * 128:(j + 1) * 128]


def _mfinal_body(a_ref, b_ref, deg_ref, out_ref):
    dis = _dis_block(deg_ref)
    hcat = jnp.concatenate([a_ref[j] for j in range(NCHUNK)], axis=1)
    out_ref[...] = dis * hcat + b_ref[...]


_deg_spec = pl.BlockSpec((RB, 128), lambda r: (r, 0))
_chunk_spec = pl.BlockSpec((NCHUNK, RB, 128), lambda r: (0, r, 0))


def _m1(x, w, deg):
    return pl.pallas_call(
        _m1_body,
        grid=(GRID_R,),
        in_specs=[
            pl.BlockSpec((RB, 256), lambda r: (r, 0)),
            pl.BlockSpec((256, 512), lambda r: (0, 0)),
            _deg_spec,
        ],
        out_specs=_chunk_spec,
        out_shape=jax.ShapeDtypeStruct((NCHUNK, NPAD, 128), jnp.float32),
    )(x, w, deg)


def _mmid(a, w, b, deg):
    return pl.pallas_call(
        _mmid_body,
        grid=(GRID_R,),
        in_specs=[
            _chunk_spec,
            pl.BlockSpec((512, 512), lambda r: (0, 0)),
            pl.BlockSpec((1, 512), lambda r: (0, 0)),
            _deg_spec,
        ],
        out_specs=_chunk_spec,
        out_shape=jax.ShapeDtypeStruct((NCHUNK, NPAD, 128), jnp.float32),
    )(a, w, b, deg)


def _mfinal(a, b, deg):
    return pl.pallas_call(
        _mfinal_body,
        grid=(GRID_R,),
        in_specs=[
            _chunk_spec,
            pl.BlockSpec((1, 512), lambda r: (0, 0)),
            _deg_spec,
        ],
        out_specs=pl.BlockSpec((RB, 512), lambda r: (r, 0)),
        out_shape=jax.ShapeDtypeStruct((N, 512), jnp.float32),
    )(a, b, deg)


def kernel(node_features, edge_index, W1, b1, W2, b2, W3, b3):
    ei = edge_index.astype(jnp.int32)
    pad = EPAD - E
    rows3 = jnp.concatenate(
        [ei[0], jnp.zeros((pad,), jnp.int32)]).reshape(NS, NBT, B)
    cols3 = jnp.concatenate(
        [ei[1], jnp.full((pad,), NPAD, jnp.int32)]).reshape(NS, NBT, B)
    ones = jnp.ones((B, 128), jnp.float32)
    zeros = jnp.zeros((RPT, 128), jnp.float32)

    deg = _sc_deg(cols3, ones, zeros)

    y1 = _m1(node_features, W1, deg)
    a1 = _sc_scatter(rows3, cols3, y1)
    y2 = _mmid(a1, W2, b1.reshape(1, 512), deg)
    a2 = _sc_scatter(rows3, cols3, y2)
    y3 = _mmid(a2, W3, b2.reshape(1, 512), deg)
    a3 = _sc_scatter(rows3, cols3, y3)
    return _mfinal(a3, b3.reshape(1, 512), deg)


# speedup vs baseline: 2.7956x; 2.7857x over previous
import functools

import jax
import jax.numpy as jnp
from jax import lax
from jax.experimental import pallas as pl
from jax.experimental.pallas import tpu as pltpu
from jax.experimental.pallas import tpu_sc as plsc

N = 10000
NPAD = 10240
E = 160000
NC = 2
NS = 16
B = 80
NBT = E // (NS * B)
HALF = NPAD // NC
ACC_R = HALF + 8
RPT = HALF // NS
NCHUNK = 4
RB = 400
GRID_R = N // RB

_mesh = plsc.VectorSubcoreMesh(core_axis_name="c", subcore_axis_name="s")


def _redirect_cols(cols_v, sc):
    base = sc * HALF

    def body(i, carry):
        for k in range(B // 16):
            c = cols_v[i, pl.ds(k * 16, 16)] - base
            ok = (c >= 0) & (c < HALF)
            cols_v[i, pl.ds(k * 16, 16)] = jnp.where(ok, c, HALF)
        return carry

    lax.fori_loop(0, NBT, body, 0)


def _deg_body(cols_hbm, ones_hbm, zeros_hbm, deg_hbm, cols_v, ones_v, dacc):
    sc = lax.axis_index("c")
    tid = lax.axis_index("s")
    pltpu.sync_copy(cols_hbm.at[tid], cols_v)
    pltpu.sync_copy(ones_hbm, ones_v)
    pltpu.sync_copy(zeros_hbm, dacc.at[pl.ds(tid * RPT, RPT)])
    _redirect_cols(cols_v, sc)
    plsc.subcore_barrier()

    def body(i, carry):
        pltpu.sync_copy(ones_v, dacc.at[cols_v.at[i]], add=True)
        return carry

    lax.fori_loop(0, NBT, body, 0)
    plsc.subcore_barrier()
    pltpu.sync_copy(dacc.at[pl.ds(tid * RPT, RPT)],
                    deg_hbm.at[pl.ds(sc * HALF + tid * RPT, RPT)])
    return None


_sc_deg = functools.partial(
    pl.kernel,
    out_type=jax.ShapeDtypeStruct((NPAD, 128), jnp.float32),
    mesh=_mesh,
    scratch_types=[
        pltpu.VMEM((NBT, B), jnp.int32),
        pltpu.VMEM((B, 128), jnp.float32),
        pltpu.VMEM_SHARED((ACC_R, 128), jnp.float32),
    ],
)(_deg_body)


def _scatter_body(rows_hbm, cols_hbm, y_hbm, out_hbm,
                  rows_v, cols_v, g0, g1, acc, s0, s1):
    sc = lax.axis_index("c")
    tid = lax.axis_index("s")
    pltpu.sync_copy(rows_hbm.at[tid], rows_v)
    pltpu.sync_copy(cols_hbm.at[tid], cols_v)
    _redirect_cols(cols_v, sc)
    for j in range(NCHUNK):
        ytab = y_hbm.at[j]
        pltpu.sync_copy(ytab.at[pl.ds(sc * HALF + tid * RPT, RPT)],
                        acc.at[pl.ds(tid * RPT, RPT)])
        plsc.subcore_barrier()

        pltpu.async_copy(ytab.at[rows_v.at[0]], g0, s0)

        def body(i, carry):
            c1 = pltpu.async_copy(ytab.at[rows_v.at[2 * i + 1]], g1, s1)
            pltpu.make_async_copy(ytab.at[rows_v.at[2 * i]], g0, s0).wait()
            pltpu.sync_copy(g0, acc.at[cols_v.at[2 * i]], add=True)
            pltpu.async_copy(ytab.at[rows_v.at[2 * i + 2]], g0, s0)
            c1.wait()
            pltpu.sync_copy(g1, acc.at[cols_v.at[2 * i + 1]], add=True)
            return carry

        lax.fori_loop(0, NBT // 2, body, 0)
        pltpu.make_async_copy(ytab.at[rows_v.at[NBT - 1]], g0, s0).wait()
        pltpu.sync_copy(g0, acc.at[cols_v.at[NBT - 1]], add=True)

        plsc.subcore_barrier()
        pltpu.sync_copy(acc.at[pl.ds(tid * RPT, RPT)],
                        out_hbm.at[j].at[pl.ds(sc * HALF + tid * RPT, RPT)])
        plsc.subcore_barrier()
    return None


_sc_scatter = functools.partial(
    pl.kernel,
    out_type=jax.ShapeDtypeStruct((NCHUNK, NPAD, 128), jnp.float32),
    mesh=_mesh,
    scratch_types=[
        pltpu.VMEM((NBT, B), jnp.int32),
        pltpu.VMEM((NBT, B), jnp.int32),
        pltpu.VMEM((B, 128), jnp.float32),
        pltpu.VMEM((B, 128), jnp.float32),
        pltpu.VMEM_SHARED((ACC_R, 128), jnp.float32),
        pltpu.SemaphoreType.DMA,
        pltpu.SemaphoreType.DMA,
    ],
)(_scatter_body)


def _dis_block(deg_ref):
    s = jnp.sum(deg_ref[...], axis=1, keepdims=True) * (1.0 / 128.0) + 1.0
    return lax.rsqrt(s)


def _m1_body(x_ref, w_ref, deg_ref, out_ref):
    dis = _dis_block(deg_ref)
    xw = jnp.dot(x_ref[...], w_ref[...], preferred_element_type=jnp.float32)
    y = dis * xw
    for j in range(NCHUNK):
        out_ref[j] = y[:, j * 128:(j + 1) * 128]


def _mmid_body(a_ref, w_ref, b_ref, deg_ref, out_ref):
    dis = _dis_block(deg_ref)
    hcat = jnp.concatenate([a_ref[j] for j in range(NCHUNK)], axis=1)
    h = dis * hcat + b_ref[...]
    xw = jnp.dot(h, w_ref[...], preferred_element_type=jnp.float32)
    y = dis * xw
    for j in range(NCHUNK):
        out_ref[j] = y[:, j * 128:(j + 1) * 128]


def _mfinal_body(a_ref, b_ref, deg_ref, out_ref):
    dis = _dis_block(deg_ref)
    hcat = jnp.concatenate([a_ref[j] for j in range(NCHUNK)], axis=1)
    out_ref[...] = dis * hcat + b_ref[...]


_deg_spec = pl.BlockSpec((RB, 128), lambda r: (r, 0))
_chunk_spec = pl.BlockSpec((NCHUNK, RB, 128), lambda r: (0, r, 0))


def _m1(x, w, deg):
    return pl.pallas_call(
        _m1_body,
        grid=(GRID_R,),
        in_specs=[
            pl.BlockSpec((RB, 256), lambda r: (r, 0)),
            pl.BlockSpec((256, 512), lambda r: (0, 0)),
            _deg_spec,
        ],
        out_specs=_chunk_spec,
        out_shape=jax.ShapeDtypeStruct((NCHUNK, NPAD, 128), jnp.float32),
    )(x, w, deg)


def _mmid(a, w, b, deg):
    return pl.pallas_call(
        _mmid_body,
        grid=(GRID_R,),
        in_specs=[
            _chunk_spec,
            pl.BlockSpec((512, 512), lambda r: (0, 0)),
            pl.BlockSpec((1, 512), lambda r: (0, 0)),
            _deg_spec,
        ],
        out_specs=_chunk_spec,
        out_shape=jax.ShapeDtypeStruct((NCHUNK, NPAD, 128), jnp.float32),
    )(a, w, b, deg)


def _mfinal(a, b, deg):
    return pl.pallas_call(
        _mfinal_body,
        grid=(GRID_R,),
        in_specs=[
            _chunk_spec,
            pl.BlockSpec((1, 512), lambda r: (0, 0)),
            _deg_spec,
        ],
        out_specs=pl.BlockSpec((RB, 512), lambda r: (r, 0)),
        out_shape=jax.ShapeDtypeStruct((N, 512), jnp.float32),
    )(a, b, deg)


def kernel(node_features, edge_index, W1, b1, W2, b2, W3, b3):
    ei = edge_index.astype(jnp.int32)
    rows3 = ei[0].reshape(NS, NBT, B)
    cols3 = ei[1].reshape(NS, NBT, B)
    ones = jnp.ones((B, 128), jnp.float32)
    zeros = jnp.zeros((RPT, 128), jnp.float32)

    deg = _sc_deg(cols3, ones, zeros)

    y1 = _m1(node_features, W1, deg)
    a1 = _sc_scatter(rows3, cols3, y1)
    y2 = _mmid(a1, W2, b1.reshape(1, 512), deg)
    a2 = _sc_scatter(rows3, cols3, y2)
    y3 = _mmid(a2, W3, b2.reshape(1, 512), deg)
    a3 = _sc_scatter(rows3, cols3, y3)
    return _mfinal(a3, b3.reshape(1, 512), deg)
